# Initial kernel scaffold; baseline (speedup 1.0000x reference)
import jax
import jax.numpy as jnp
from jax.experimental import pallas as pl


def kernel(feature, edge_index, edge_weight, W, b):
    raise NotImplementedError("write your pallas kernel here")


# speedup vs baseline: 3.5577x; 3.5577x over previous
import functools

import jax
import jax.numpy as jnp
from jax import lax
from jax.experimental import pallas as pl
from jax.experimental.pallas import tpu as pltpu
from jax.experimental.pallas import tpu_sc as plsc

_NC = 2
_NS = 16
_LANES = 16
_CH = 128


def _matmul_relu(feature, W):
    n, f = feature.shape
    d = W.shape[1]
    blk = 1000

    def body(f_ref, w_ref, o_ref):
        o_ref[...] = jnp.maximum(
            jnp.dot(f_ref[...], w_ref[...], preferred_element_type=jnp.float32),
            0.0,
        )

    return pl.pallas_call(
        body,
        grid=(n // blk,),
        in_specs=[
            pl.BlockSpec((blk, f), lambda i: (i, 0)),
            pl.BlockSpec((f, d), lambda i: (0, 0)),
        ],
        out_specs=pl.BlockSpec((blk, d), lambda i: (i, 0)),
        out_shape=jax.ShapeDtypeStruct((n, d), jnp.float32),
    )(feature, W)


def _combine_bias(partials, b2d):
    nc, n, d = partials.shape
    blk = 1000

    def body(p_ref, b_ref, o_ref):
        o_ref[...] = p_ref[0] + p_ref[1] + b_ref[...]

    return pl.pallas_call(
        body,
        grid=(n // blk,),
        in_specs=[
            pl.BlockSpec((nc, blk, d), lambda i: (0, i, 0)),
            pl.BlockSpec((1, d), lambda i: (0, 0)),
        ],
        out_specs=pl.BlockSpec((blk, d), lambda i: (i, 0)),
        out_shape=jax.ShapeDtypeStruct((n, d), jnp.float32),
    )(partials, b2d)


def _make_spmm(n_nodes, n_edges, d):
    chunks_total = n_edges // _CH
    chunks_per_core = chunks_total // _NC
    iters = (chunks_per_core + _NS - 1) // _NS
    row_chunks_full = n_nodes // _CH
    row_rem = n_nodes - row_chunks_full * _CH
    row_chunks = row_chunks_full + (1 if row_rem else 0)
    row_iters = (row_chunks + _NS - 1) // _NS

    mesh = plsc.VectorSubcoreMesh(core_axis_name="c", subcore_axis_name="s")

    @functools.partial(
        pl.kernel,
        mesh=mesh,
        out_type=jax.ShapeDtypeStruct((_NC, n_nodes, d), jnp.float32),
        scratch_types=[
            pltpu.VMEM((_CH,), jnp.int32),
            pltpu.VMEM((1, _CH), jnp.int32),
            pltpu.VMEM((_CH, _LANES), jnp.float32),
            pltpu.VMEM((_CH, d), jnp.float32),
            pltpu.VMEM_SHARED((n_nodes, d), jnp.float32),
            pltpu.SemaphoreType.DMA,
        ],
    )
    def spmm(support_hbm, src_hbm, dst_hbm, ew_hbm, out_hbm,
             src_v, dst_v, ew_v, rows_v, acc_sh, sem):
        cid = lax.axis_index("c")
        sid = lax.axis_index("s")

        def zrow(j, carry):
            for k in range(d // _LANES):
                rows_v[j, pl.ds(k * _LANES, _LANES)] = jnp.zeros(
                    (_LANES,), jnp.float32)
            return carry

        lax.fori_loop(0, _CH, zrow, 0)
        for i in range(row_iters):
            j = sid + i * _NS

            r0 = pl.multiple_of(j * _CH, _CH)

            @pl.when(j < row_chunks_full)
            def _():
                pltpu.sync_copy(rows_v, acc_sh.at[pl.ds(r0, _CH)])

            if row_rem:
                @pl.when(j == row_chunks_full)
                def _():
                    pltpu.sync_copy(
                        rows_v.at[pl.ds(0, row_rem)],
                        acc_sh.at[pl.ds(row_chunks_full * _CH, row_rem)])
        plsc.subcore_barrier()

        def body(i, carry):
            c = sid + i * _NS

            @pl.when(c < chunks_per_core)
            def _():
                base = (cid * chunks_per_core + c) * _CH
                pltpu.sync_copy(src_hbm.at[pl.ds(base, _CH)], src_v)
                pltpu.sync_copy(dst_hbm.at[pl.ds(base, _CH)], dst_v.at[0])
                pltpu.sync_copy(ew_hbm.at[pl.ds(base, _CH)], ew_v)
                pltpu.async_copy(support_hbm.at[src_v], rows_v, sem).wait()

                def scale(j, c2):
                    w = ew_v[j]
                    for k in range(d // _LANES):
                        sl = pl.ds(k * _LANES, _LANES)
                        rows_v[j, sl] = rows_v[j, sl] * w
                    return c2

                lax.fori_loop(0, _CH, scale, 0)
                pltpu.sync_copy(rows_v, acc_sh.at[dst_v.at[0]], add=True)

            return carry

        lax.fori_loop(0, iters, body, 0)
        plsc.subcore_barrier()

        for i in range(row_iters):
            j = sid + i * _NS
            r0 = pl.multiple_of(j * _CH, _CH)

            @pl.when(j < row_chunks_full)
            def _():
                pltpu.sync_copy(acc_sh.at[pl.ds(r0, _CH)],
                                out_hbm.at[cid, pl.ds(r0, _CH)])

            if row_rem:
                @pl.when(j == row_chunks_full)
                def _():
                    rr = row_chunks_full * _CH
                    pltpu.sync_copy(acc_sh.at[pl.ds(rr, row_rem)],
                                    out_hbm.at[cid, pl.ds(rr, row_rem)])

    return spmm


def kernel(feature, edge_index, edge_weight, W, b):
    n, f = feature.shape
    d = W.shape[1]
    e = edge_weight.shape[0]

    support = _matmul_relu(feature, W)
    src = edge_index[0]
    dst = edge_index[1]
    ew_rep = jnp.broadcast_to(edge_weight[:, None], (e, _LANES))
    partials = _make_spmm(n, e, d)(support, src, dst, ew_rep)
    return _combine_bias(partials, b.reshape(1, d))


# speedup vs baseline: 3.6314x; 1.0207x over previous
import functools

import jax
import jax.numpy as jnp
from jax import lax
from jax.experimental import pallas as pl
from jax.experimental.pallas import tpu as pltpu
from jax.experimental.pallas import tpu_sc as plsc

_NC = 2
_NS = 16
_NW = _NC * _NS
_LANES = 16
_CH = 128


def _matmul_relu(feature, W):
    n, f = feature.shape
    d = W.shape[1]
    blk = 1000

    def body(f_ref, w_ref, o_ref):
        o_ref[...] = jnp.maximum(
            jnp.dot(f_ref[...], w_ref[...], preferred_element_type=jnp.float32),
            0.0,
        )

    return pl.pallas_call(
        body,
        grid=(n // blk,),
        in_specs=[
            pl.BlockSpec((blk, f), lambda i: (i, 0)),
            pl.BlockSpec((f, d), lambda i: (0, 0)),
        ],
        out_specs=pl.BlockSpec((blk, d), lambda i: (i, 0)),
        out_shape=jax.ShapeDtypeStruct((n, d), jnp.float32),
    )(feature, W)


def _combine_bias(partials, b2d):
    nc, n, d = partials.shape
    blk = 1000

    def body(p_ref, b_ref, o_ref):
        o_ref[...] = p_ref[0] + p_ref[1] + b_ref[...]

    return pl.pallas_call(
        body,
        grid=(n // blk,),
        in_specs=[
            pl.BlockSpec((nc, blk, d), lambda i: (0, i, 0)),
            pl.BlockSpec((1, d), lambda i: (0, 0)),
        ],
        out_specs=pl.BlockSpec((blk, d), lambda i: (i, 0)),
        out_shape=jax.ShapeDtypeStruct((n, d), jnp.float32),
    )(partials, b2d)


def _make_spmm(n_nodes, e_pad, d):
    chunks_per_tile = e_pad // (_NW * _CH)
    ept = chunks_per_tile * _CH
    nphases = 2
    cpp = chunks_per_tile // nphases
    epp = cpp * _CH
    half = cpp // 2
    row_chunks_full = n_nodes // _CH
    row_rem = n_nodes - row_chunks_full * _CH
    row_chunks = row_chunks_full + (1 if row_rem else 0)
    row_iters = (row_chunks + _NS - 1) // _NS

    mesh = plsc.VectorSubcoreMesh(core_axis_name="c", subcore_axis_name="s")

    @functools.partial(
        pl.kernel,
        mesh=mesh,
        out_type=jax.ShapeDtypeStruct((_NC, n_nodes, d), jnp.float32),
        scratch_types=[
            pltpu.VMEM((epp,), jnp.int32),
            pltpu.VMEM((cpp, _CH), jnp.int32),
            pltpu.VMEM((epp,), jnp.float32),
            pltpu.VMEM((_CH, d), jnp.float32),
            pltpu.VMEM((_CH, d), jnp.float32),
            pltpu.VMEM_SHARED((n_nodes, d), jnp.float32),
            pltpu.SemaphoreType.DMA,
            pltpu.SemaphoreType.DMA,
            pltpu.SemaphoreType.DMA,
            pltpu.SemaphoreType.DMA,
        ],
    )
    def spmm(support_hbm, src_hbm, dst_hbm, ew_hbm, out_hbm,
             src_v, dst_v, ew_v, rows0, rows1, acc_sh,
             gsem0, gsem1, ssem0, ssem1):
        cid = lax.axis_index("c")
        sid = lax.axis_index("s")
        wid = cid * _NS + sid

        def zrow(j, carry):
            for k in range(d // _LANES):
                rows0[j, pl.ds(k * _LANES, _LANES)] = jnp.zeros(
                    (_LANES,), jnp.float32)
            return carry

        lax.fori_loop(0, _CH, zrow, 0)
        for i in range(row_iters):
            j = sid + i * _NS
            r0 = pl.multiple_of(j * _CH, _CH)

            @pl.when(j < row_chunks_full)
            def _():
                pltpu.sync_copy(rows0, acc_sh.at[pl.ds(r0, _CH)])

            if row_rem:
                @pl.when(j == row_chunks_full)
                def _():
                    pltpu.sync_copy(
                        rows0.at[pl.ds(0, row_rem)],
                        acc_sh.at[pl.ds(row_chunks_full * _CH, row_rem)])
        plsc.subcore_barrier()

        def gather(c, rows, sem):
            return pltpu.async_copy(
                support_hbm.at[src_v.at[pl.ds(c * _CH, _CH)]], rows, sem)

        def scatter(c, rows, sem):
            return pltpu.async_copy(rows, acc_sh.at[dst_v.at[c]], sem,
                                    add=True)

        bcast_dnums = lax.GatherDimensionNumbers(
            offset_dims=(), collapsed_slice_dims=(0,), start_index_map=(0,))

        def scale(c, rows):
            def grp(g, carry):
                eww = ew_v[pl.ds(c * _CH + g * _LANES, _LANES)]
                for jj in range(_LANES):
                    bw = lax.gather(
                        eww, jnp.full((_LANES, 1), jj, jnp.int32),
                        bcast_dnums, slice_sizes=(1,),
                        mode=lax.GatherScatterMode.PROMISE_IN_BOUNDS)
                    j = g * _LANES + jj
                    for k in range(d // _LANES):
                        sl = pl.ds(k * _LANES, _LANES)
                        rows[j, sl] = rows[j, sl] * bw
                return carry

            lax.fori_loop(0, _CH // _LANES, grp, 0)

        def gwait(c, rows, sem):
            pltpu.make_async_copy(
                support_hbm.at[src_v.at[pl.ds(c * _CH, _CH)]], rows, sem
            ).wait()

        def swait(rows, sem):
            pltpu.make_async_copy(rows, acc_sh.at[dst_v.at[0]], sem).wait()

        for phase in range(nphases):
            ebase = pl.multiple_of(wid * ept + phase * epp, _CH)
            pltpu.sync_copy(src_hbm.at[pl.ds(ebase, epp)], src_v)
            pltpu.sync_copy(
                dst_hbm.at[pl.ds(wid * chunks_per_tile + phase * cpp, cpp)],
                dst_v)
            pltpu.sync_copy(ew_hbm.at[pl.ds(ebase, epp)], ew_v)
            gather(0, rows0, gsem0)

            def body(i2, carry):
                a = 2 * i2
                b = a + 1
                gwait(a, rows0, gsem0)

                @pl.when(i2 > 0)
                def _():
                    swait(rows1, ssem1)

                gather(b, rows1, gsem1)
                scale(a, rows0)
                scatter(a, rows0, ssem0)
                gwait(b, rows1, gsem1)
                scale(b, rows1)
                swait(rows0, ssem0)

                @pl.when(i2 < half - 1)
                def _():
                    gather(a + 2, rows0, gsem0)

                scatter(b, rows1, ssem1)
                return carry

            lax.fori_loop(0, half, body, 0)
            swait(rows1, ssem1)
        plsc.subcore_barrier()

        for i in range(row_iters):
            j = sid + i * _NS
            r0 = pl.multiple_of(j * _CH, _CH)

            @pl.when(j < row_chunks_full)
            def _():
                pltpu.sync_copy(acc_sh.at[pl.ds(r0, _CH)],
                                out_hbm.at[cid, pl.ds(r0, _CH)])

            if row_rem:
                @pl.when(j == row_chunks_full)
                def _():
                    rr = row_chunks_full * _CH
                    pltpu.sync_copy(acc_sh.at[pl.ds(rr, row_rem)],
                                    out_hbm.at[cid, pl.ds(rr, row_rem)])

    return spmm


def kernel(feature, edge_index, edge_weight, W, b):
    n, f = feature.shape
    d = W.shape[1]
    e = edge_weight.shape[0]

    support = _matmul_relu(feature, W)

    grain = _NW * _CH
    e_pad = ((e + grain - 1) // grain) * grain
    if e_pad % (2 * grain):
        e_pad += grain
    pad = e_pad - e
    src = jnp.pad(edge_index[0], (0, pad))
    dst = jnp.pad(edge_index[1], (0, pad))
    ew = jnp.pad(edge_weight, (0, pad))
    dst2d = dst.reshape(e_pad // _CH, _CH)

    partials = _make_spmm(n, e_pad, d)(support, src, dst2d, ew)
    return _combine_bias(partials, b.reshape(1, d))


# speedup vs baseline: 9.8781x; 2.7202x over previous
import functools

import jax
import jax.numpy as jnp
from jax import lax
from jax.experimental import pallas as pl
from jax.experimental.pallas import tpu as pltpu
from jax.experimental.pallas import tpu_sc as plsc

_NC = 2
_NS = 16
_NW = _NC * _NS
_LANES = 16
_CH = 128


def _matmul_relu(feature, W):
    n, f = feature.shape
    d = W.shape[1]
    blk = 1000

    def body(f_ref, w_ref, o_ref):
        o_ref[...] = jnp.maximum(
            jnp.dot(f_ref[...], w_ref[...], preferred_element_type=jnp.float32),
            0.0,
        )

    return pl.pallas_call(
        body,
        grid=(n // blk,),
        in_specs=[
            pl.BlockSpec((blk, f), lambda i: (i, 0)),
            pl.BlockSpec((f, d), lambda i: (0, 0)),
        ],
        out_specs=pl.BlockSpec((blk, d), lambda i: (i, 0)),
        out_shape=jax.ShapeDtypeStruct((n, d), jnp.float32),
    )(feature, W)


def _combine_bias(partials, b2d):
    nc, n, d = partials.shape
    blk = 1000

    def body(p_ref, b_ref, o_ref):
        o_ref[...] = p_ref[0] + p_ref[1] + b_ref[...]

    return pl.pallas_call(
        body,
        grid=(n // blk,),
        in_specs=[
            pl.BlockSpec((nc, blk, d), lambda i: (0, i, 0)),
            pl.BlockSpec((1, d), lambda i: (0, 0)),
        ],
        out_specs=pl.BlockSpec((blk, d), lambda i: (i, 0)),
        out_shape=jax.ShapeDtypeStruct((n, d), jnp.float32),
    )(partials, b2d)


def _make_spmm(n_nodes, e_pad, d):
    chunks_per_tile = e_pad // (_NW * _CH)
    ept = chunks_per_tile * _CH
    nphases = 2
    cpp = chunks_per_tile // nphases
    epp = cpp * _CH
    half = cpp // 2
    row_chunks_full = n_nodes // _CH
    row_rem = n_nodes - row_chunks_full * _CH
    row_chunks = row_chunks_full + (1 if row_rem else 0)
    row_iters = (row_chunks + _NS - 1) // _NS

    mesh = plsc.VectorSubcoreMesh(core_axis_name="c", subcore_axis_name="s")

    @functools.partial(
        pl.kernel,
        mesh=mesh,
        out_type=jax.ShapeDtypeStruct((_NC, n_nodes, d), jnp.float32),
        scratch_types=[
            pltpu.VMEM((epp,), jnp.int32),
            pltpu.VMEM((cpp, _CH), jnp.int32),
            pltpu.VMEM((epp,), jnp.float32),
            pltpu.VMEM((_CH, d), jnp.float32),
            pltpu.VMEM((_CH, d), jnp.float32),
            pltpu.VMEM_SHARED((n_nodes, d), jnp.float32),
            pltpu.SemaphoreType.DMA,
            pltpu.SemaphoreType.DMA,
            pltpu.SemaphoreType.DMA,
            pltpu.SemaphoreType.DMA,
        ],
    )
    def spmm(support_hbm, src_hbm, dst_hbm, ew_hbm, out_hbm,
             src_v, dst_v, ew_v, rows0, rows1, acc_sh,
             gsem0, gsem1, ssem0, ssem1):
        cid = lax.axis_index("c")
        sid = lax.axis_index("s")
        wid = cid * _NS + sid

        def zrow(j, carry):
            for k in range(d // _LANES):
                rows0[j, pl.ds(k * _LANES, _LANES)] = jnp.zeros(
                    (_LANES,), jnp.float32)
            return carry

        lax.fori_loop(0, _CH, zrow, 0)
        for i in range(row_iters):
            j = sid + i * _NS
            r0 = pl.multiple_of(j * _CH, _CH)

            @pl.when(j < row_chunks_full)
            def _():
                pltpu.sync_copy(rows0, acc_sh.at[pl.ds(r0, _CH)])

            if row_rem:
                @pl.when(j == row_chunks_full)
                def _():
                    pltpu.sync_copy(
                        rows0.at[pl.ds(0, row_rem)],
                        acc_sh.at[pl.ds(row_chunks_full * _CH, row_rem)])
        plsc.subcore_barrier()

        def gather(c, rows, sem):
            return pltpu.async_copy(
                support_hbm.at[src_v.at[pl.ds(c * _CH, _CH)]], rows, sem)

        def scatter(c, rows, sem):
            return pltpu.async_copy(rows, acc_sh.at[dst_v.at[c]], sem,
                                    add=True)

        bcast_dnums = lax.GatherDimensionNumbers(
            offset_dims=(), collapsed_slice_dims=(0,), start_index_map=(0,))

        def scale(c, rows):
            def grp(g, carry):
                eww = ew_v[pl.ds(c * _CH + g * _LANES, _LANES)]
                for jj in range(_LANES):
                    bw = lax.gather(
                        eww, jnp.full((_LANES, 1), jj, jnp.int32),
                        bcast_dnums, slice_sizes=(1,),
                        mode=lax.GatherScatterMode.PROMISE_IN_BOUNDS)
                    j = g * _LANES + jj
                    for k in range(d // _LANES):
                        sl = pl.ds(k * _LANES, _LANES)
                        rows[j, sl] = rows[j, sl] * bw
                return carry

            lax.fori_loop(0, _CH // _LANES, grp, 0)

        def gwait(c, rows, sem):
            pltpu.make_async_copy(
                support_hbm.at[src_v.at[pl.ds(c * _CH, _CH)]], rows, sem
            ).wait()

        def swait(rows, sem):
            pltpu.make_async_copy(rows, acc_sh.at[dst_v.at[0]], sem).wait()

        for phase in range(nphases):
            ebase = pl.multiple_of(wid * ept + phase * epp, _CH)
            pltpu.sync_copy(src_hbm.at[pl.ds(ebase, epp)], src_v)
            pltpu.sync_copy(
                dst_hbm.at[pl.ds(wid * chunks_per_tile + phase * cpp, cpp)],
                dst_v)
            pltpu.sync_copy(ew_hbm.at[pl.ds(ebase, epp)], ew_v)
            gather(0, rows0, gsem0)

            def body(i2, carry):
                a = 2 * i2
                b = a + 1
                gwait(a, rows0, gsem0)

                @pl.when(i2 > 0)
                def _():
                    swait(rows1, ssem1)

                gather(b, rows1, gsem1)
                scale(a, rows0)
                scatter(a, rows0, ssem0)
                gwait(b, rows1, gsem1)
                scale(b, rows1)
                swait(rows0, ssem0)

                @pl.when(i2 < half - 1)
                def _():
                    gather(a + 2, rows0, gsem0)

                scatter(b, rows1, ssem1)
                return carry

            lax.fori_loop(0, half, body, 0)
            swait(rows1, ssem1)
        plsc.subcore_barrier()

        for i in range(row_iters):
            j = sid + i * _NS
            r0 = pl.multiple_of(j * _CH, _CH)

            @pl.when(j < row_chunks_full)
            def _():
                pltpu.sync_copy(acc_sh.at[pl.ds(r0, _CH)],
                                out_hbm.at[cid, pl.ds(r0, _CH)])

            if row_rem:
                @pl.when(j == row_chunks_full)
                def _():
                    rr = row_chunks_full * _CH
                    pltpu.sync_copy(acc_sh.at[pl.ds(rr, row_rem)],
                                    out_hbm.at[cid, pl.ds(rr, row_rem)])

    return spmm


def kernel(feature, edge_index, edge_weight, W, b):
    n, f = feature.shape
    d = W.shape[1]
    e = edge_weight.shape[0]

    support = _matmul_relu(feature, W)

    grain = _NW * _CH
    e_pad = ((e + grain - 1) // grain) * grain
    if e_pad % (2 * grain):
        e_pad += grain
    pad = e_pad - e
    spread = jnp.arange(pad, dtype=jnp.int32) % n
    src = jnp.concatenate([edge_index[0], spread])
    dst = jnp.concatenate([edge_index[1], spread])
    ew = jnp.pad(edge_weight, (0, pad))
    dst2d = dst.reshape(e_pad // _CH, _CH)

    partials = _make_spmm(n, e_pad, d)(support, src, dst2d, ew)
    return _combine_bias(partials, b.reshape(1, d))
